# baseline (speedup 1.0000x reference)
import functools

import jax
import jax.numpy as jnp
from jax import lax
from jax.experimental import pallas as pl
from jax.experimental.pallas import tpu as pltpu
from jax.experimental.pallas import tpu_sc as plsc

N = 10000
E = 320000
H = 8
CW = 32
MW = CW + 16
NSUB = 16
NCORE = 2
EB = 80
EPT = E // NSUB
ZB = 80
NZBLK = N // ZB
QNB = 50
NBLK = 10
BLK = N // NBLK


def _proj_body(nsplit, x_ref, w_ref, aw_ref, h_ref, al_ref):
    q = pl.program_id(1)
    h = jnp.dot(x_ref[...], w_ref[0], preferred_element_type=jnp.float32)
    h_ref[0] = h
    al = jnp.dot(h, aw_ref[0], preferred_element_type=jnp.float32)
    if nsplit == 1:
        al_ref[0] = al
    else:
        @pl.when(q % nsplit == 0)
        def _():
            al_ref[0] = al

        @pl.when(q % nsplit != 0)
        def _():
            al_ref[0] += al


def _proj(x, w, a_s, a_d):
    din, _, c = w.shape
    nsplit = c // CW
    nq = H * nsplit
    wf = w.reshape(din, H, nsplit, CW).transpose(1, 2, 0, 3).reshape(nq, din, CW)
    aw = jnp.stack([a_s, a_d], axis=-1).reshape(H, nsplit, CW, 2).reshape(nq, CW, 2)
    return pl.pallas_call(
        functools.partial(_proj_body, nsplit),
        grid=(NBLK, nq),
        in_specs=[
            pl.BlockSpec((BLK, din), lambda i, q: (i, 0)),
            pl.BlockSpec((1, din, CW), lambda i, q: (q, 0, 0)),
            pl.BlockSpec((1, CW, 2), lambda i, q: (q, 0, 0)),
        ],
        out_specs=[
            pl.BlockSpec((1, BLK, CW), lambda i, q: (q, i, 0)),
            pl.BlockSpec((1, BLK, 2), lambda i, q: (q // nsplit, i, 0)),
        ],
        out_shape=[
            jax.ShapeDtypeStruct((nq, N, CW), jnp.float32),
            jax.ShapeDtypeStruct((H, N, 2), jnp.float32),
        ],
    )(x, wf, aw)


def _epilogue(agg_ref, b_ref, nqp):
    parts = []
    for qq in range(nqp):
        a = agg_ref[qq]
        den = jnp.broadcast_to(a[:, CW:CW + 1], (BLK, CW))
        parts.append(a[:, 0:CW] / (den + 1e-9))
    out = jnp.concatenate(parts, axis=-1) + b_ref[...]
    return jnp.maximum(out, 0.0)


def _fproj_body(nsplit, nqp, agg_ref, b_ref, w_ref, aw_ref, h_ref, al_ref):
    q = pl.program_id(1)
    x = _epilogue(agg_ref, b_ref, nqp)
    h = jnp.dot(x, w_ref[0], preferred_element_type=jnp.float32)
    h_ref[0] = h
    al = jnp.dot(h, aw_ref[0], preferred_element_type=jnp.float32)
    if nsplit == 1:
        al_ref[0] = al
    else:
        @pl.when(q % nsplit == 0)
        def _():
            al_ref[0] = al

        @pl.when(q % nsplit != 0)
        def _():
            al_ref[0] += al


def _fproj(agg, b, w, a_s, a_d):
    din, _, c = w.shape
    nqp = agg.shape[0]
    nsplit = c // CW
    nq = H * nsplit
    wf = w.reshape(din, H, nsplit, CW).transpose(1, 2, 0, 3).reshape(nq, din, CW)
    aw = jnp.stack([a_s, a_d], axis=-1).reshape(H, nsplit, CW, 2).reshape(nq, CW, 2)
    return pl.pallas_call(
        functools.partial(_fproj_body, nsplit, nqp),
        grid=(NBLK, nq),
        in_specs=[
            pl.BlockSpec((nqp, BLK, MW), lambda i, q: (0, i, 0)),
            pl.BlockSpec((1, din), lambda i, q: (0, 0)),
            pl.BlockSpec((1, din, CW), lambda i, q: (q, 0, 0)),
            pl.BlockSpec((1, CW, 2), lambda i, q: (q, 0, 0)),
        ],
        out_specs=[
            pl.BlockSpec((1, BLK, CW), lambda i, q: (q, i, 0)),
            pl.BlockSpec((1, BLK, 2), lambda i, q: (q // nsplit, i, 0)),
        ],
        out_shape=[
            jax.ShapeDtypeStruct((nq, N, CW), jnp.float32),
            jax.ShapeDtypeStruct((H, N, 2), jnp.float32),
        ],
    )(agg, b.reshape(1, din), wf, aw)


def _prep_body(al_ref, a_ref):
    al = al_ref[0]
    amax = jnp.max(al[:, 0:1], axis=0, keepdims=True)
    a_ref[...] = jnp.broadcast_to(amax, (1, 1, 16))


def _prep(al):
    return pl.pallas_call(
        _prep_body,
        grid=(H,),
        in_specs=[pl.BlockSpec((1, N, 2), lambda h: (h, 0, 0))],
        out_specs=pl.BlockSpec((1, 1, 16), lambda h: (h, 0, 0)),
        out_shape=jax.ShapeDtypeStruct((H, 1, 16), jnp.float32),
    )(al)


def _fpool_body(agg1_ref, b1_ref, agg2_ref, b2_ref, o_ref):
    i = pl.program_id(0)
    x1 = _epilogue(agg1_ref, b1_ref, agg1_ref.shape[0])
    x2 = _epilogue(agg2_ref, b2_ref, agg2_ref.shape[0])
    s = jnp.concatenate([jnp.sum(x1, axis=0), jnp.sum(x2, axis=0)], axis=-1)

    @pl.when(i == 0)
    def _():
        o_ref[...] = jnp.zeros_like(o_ref)

    o_ref[0, :] += s


def _fpool(agg1, b1, agg2, b2):
    nq1 = agg1.shape[0]
    nq2 = agg2.shape[0]
    d = (nq1 + nq2) * CW
    out = pl.pallas_call(
        _fpool_body,
        grid=(NBLK,),
        in_specs=[
            pl.BlockSpec((nq1, BLK, MW), lambda i: (0, i, 0)),
            pl.BlockSpec((1, nq1 * CW), lambda i: (0, 0)),
            pl.BlockSpec((nq2, BLK, MW), lambda i: (0, i, 0)),
            pl.BlockSpec((1, nq2 * CW), lambda i: (0, 0)),
        ],
        out_specs=pl.BlockSpec((1, d), lambda i: (0, 0)),
        out_shape=jax.ShapeDtypeStruct((1, d), jnp.float32),
    )(agg1, b1.reshape(1, nq1 * CW), agg2, b2.reshape(1, nq2 * CW))
    return out[0]


NB = EPT // EB
PRO = 4


def _chunk_section(ch, hh, src_hbm, dst_hbm, t_hbm, a_hbm, h_hbm, out_hbm,
                   tloc, abuf, srcl, dstl, rows, msg, pb, zbuf, agg_sh, sid,
                   semg):
    pltpu.sync_copy(t_hbm.at[hh], tloc)
    pltpu.sync_copy(a_hbm.at[hh], abuf)
    for k in range(-(-NZBLK // NSUB)):
        zb = sid + k * NSUB

        @pl.when(zb < NZBLK)
        def _():
            pltpu.sync_copy(zbuf, agg_sh.at[pl.ds(zb * ZB, ZB)])
    plsc.subcore_barrier()

    @pl.loop(0, NB // QNB)
    def _(half):
        pltpu.sync_copy(src_hbm.at[sid].at[pl.ds(half * QNB, QNB)], srcl)
        pltpu.sync_copy(dst_hbm.at[sid].at[pl.ds(half * QNB, QNB)], dstl)

        def gsrc(lb):
            return h_hbm.at[ch].at[srcl.at[lb]]

        def sdst(lb):
            return agg_sh.at[dstl.at[lb]]

        pltpu.async_copy(gsrc(0), rows.at[0], semg.at[0])

        @pl.loop(0, QNB)
        def _(lb):
            r = lax.rem(lb, 2)
            nxt = 1 - r
            pltpu.make_async_copy(gsrc(lb), rows.at[r], semg.at[r]).wait()

            @pl.when(lb + 1 < QNB)
            def _():
                pltpu.async_copy(gsrc(lb + 1), rows.at[nxt], semg.at[nxt])

            av = abuf[0, pl.ds(0, 16)]
            for g in range(EB // 16):
                sv = srcl[lb, pl.ds(g * 16, 16)]
                dv = dstl[lb, pl.ds(g * 16, 16)]
                als = plsc.load_gather(
                    tloc, [sv, jnp.zeros((16,), jnp.int32)])
                ald = plsc.load_gather(
                    tloc, [dv, jnp.full((16,), 1, jnp.int32)])
                q = als + ald
                e = jnp.maximum(q, 0.2 * q)
                t = av + ald
                mm = jnp.maximum(t, 0.2 * t)
                p = jnp.exp(e - mm)
                pb[pl.ds(g * 16, 16)] = p
                plsc.store_scatter(
                    msg,
                    [lax.iota(jnp.int32, 16) + g * 16,
                     jnp.full((16,), CW, jnp.int32)],
                    p)

            @pl.loop(0, EB)
            def _(j):
                pj = plsc.load_gather(pb, [jnp.full((16,), j, jnp.int32)])
                for v in range(CW // 16):
                    msg[j, pl.ds(v * 16, 16)] = (
                        rows[r, j, pl.ds(v * 16, 16)] * pj)

            pltpu.sync_copy(msg, sdst(lb), add=True)

    plsc.subcore_barrier()
    for k in range(-(-NZBLK // NSUB)):
        zb = sid + k * NSUB

        @pl.when(zb < NZBLK)
        def _():
            pltpu.sync_copy(agg_sh.at[pl.ds(zb * ZB, ZB)],
                            out_hbm.at[ch].at[pl.ds(zb * ZB, ZB)])
    plsc.subcore_barrier()


def _edge_pair_body(nqa, nqb, src_hbm, dst_hbm, ta_hbm, aa_hbm, tb_hbm,
                    ab_hbm, ha_hbm, hb_hbm, outa_hbm, outb_hbm, tloc, abuf,
                    srcl, dstl, rows, msg, pb, zbuf, agg_sh, semg):
    cid = lax.axis_index("c")
    sid = lax.axis_index("s")
    zero16 = jnp.zeros((16,), jnp.float32)
    nsa = nqa // H

    @pl.loop(0, ZB)
    def _(r):
        for v in range(MW // 16):
            zbuf[r, pl.ds(v * 16, 16)] = zero16

    @pl.loop(0, EB)
    def _(r):
        msg[r, pl.ds(CW, 16)] = zero16

    for k in range(nqa // NCORE):
        ch = cid * (nqa // NCORE) + k
        hh = cid * (nqa // NCORE // nsa) + k // nsa
        _chunk_section(ch, hh, src_hbm, dst_hbm, ta_hbm, aa_hbm, ha_hbm,
                       outa_hbm, tloc, abuf, srcl, dstl, rows, msg, pb,
                       zbuf, agg_sh, sid, semg)
    for k in range(nqb // NCORE):
        ch = cid * (nqb // NCORE) + k
        _chunk_section(ch, ch, src_hbm, dst_hbm, tb_hbm, ab_hbm, hb_hbm,
                       outb_hbm, tloc, abuf, srcl, dstl, rows, msg, pb,
                       zbuf, agg_sh, sid, semg)


def _edge_pair(src3d, dst3d, ta, aa, tb, ab, ha, hb):
    nqa = ha.shape[0]
    nqb = hb.shape[0]
    mesh = plsc.VectorSubcoreMesh(core_axis_name="c", subcore_axis_name="s")
    fn = pl.kernel(
        functools.partial(_edge_pair_body, nqa, nqb),
        out_type=[
            jax.ShapeDtypeStruct((nqa, N, MW), jnp.float32),
            jax.ShapeDtypeStruct((nqb, N, MW), jnp.float32),
        ],
        mesh=mesh,
        compiler_params=pltpu.CompilerParams(
            needs_layout_passes=False, use_tc_tiling_on_sc=False),
        scratch_types=[
            pltpu.VMEM((N, 2), jnp.float32),
            pltpu.VMEM((1, 16), jnp.float32),
            pltpu.VMEM((QNB, EB), jnp.int32),
            pltpu.VMEM((QNB, EB), jnp.int32),
            pltpu.VMEM((2, EB, CW), jnp.float32),
            pltpu.VMEM((EB, MW), jnp.float32),
            pltpu.VMEM((EB,), jnp.float32),
            pltpu.VMEM((ZB, MW), jnp.float32),
            pltpu.VMEM_SHARED((N, MW), jnp.float32),
            pltpu.SemaphoreType.DMA((2,)),
        ],
    )
    return fn(src3d, dst3d, ta, aa, tb, ab, ha, hb)


def _edges(al_a, al_b, ha, hb, src, dst):
    return _edge_pair(src, dst, al_a, _prep(al_a), al_b, _prep(al_b), ha, hb)


def kernel(x, W1, a_src1, a_dst1, b1, W2, a_src2, a_dst2, b2, W3, a_src3, a_dst3, b3,
           W4, a_src4, a_dst4, b4, W5, a_src5, a_dst5, b5, W6, a_src6, a_dst6, b6,
           edge_index):
    src = edge_index[0].reshape(NSUB, NB, EB)
    dst = edge_index[1].reshape(NSUB, NB, EB)
    h1, al1 = _proj(x, W1, a_src1, a_dst1)
    h4, al4 = _proj(x, W4, a_src4, a_dst4)
    agg1, agg4 = _edges(al1, al4, h1, h4, src, dst)
    h2, al2 = _fproj(agg1, b1, W2, a_src2, a_dst2)
    h5, al5 = _fproj(agg4, b4, W5, a_src5, a_dst5)
    agg2, agg5 = _edges(al2, al5, h2, h5, src, dst)
    h3, al3 = _fproj(agg2, b2, W3, a_src3, a_dst3)
    h6, al6 = _fproj(agg5, b5, W6, a_src6, a_dst6)
    agg3, agg6 = _edges(al3, al6, h3, h6, src, dst)
    return _fpool(agg3, b3, agg6, b6)


# speedup vs baseline: 1.0795x; 1.0795x over previous
import functools

import jax
import jax.numpy as jnp
from jax import lax
from jax.experimental import pallas as pl
from jax.experimental.pallas import tpu as pltpu
from jax.experimental.pallas import tpu_sc as plsc

N = 10000
E = 320000
H = 8
CW = 32
MW = CW + 16
NSUB = 16
NCORE = 2
EB = 80
EPT = E // NSUB
ZB = 80
NZBLK = N // ZB
QNB = 50
NBLK = 10
BLK = N // NBLK


def _proj_body(nsplit, x_ref, w_ref, aw_ref, h_ref, al_ref):
    q = pl.program_id(1)
    h = jnp.dot(x_ref[...], w_ref[0], preferred_element_type=jnp.float32)
    h_ref[0] = h
    al = jnp.dot(h, aw_ref[0], preferred_element_type=jnp.float32)
    if nsplit == 1:
        al_ref[0] = al
    else:
        @pl.when(q % nsplit == 0)
        def _():
            al_ref[0] = al

        @pl.when(q % nsplit != 0)
        def _():
            al_ref[0] += al


def _proj(x, w, a_s, a_d):
    din, _, c = w.shape
    nsplit = c // CW
    nq = H * nsplit
    wf = w.reshape(din, H, nsplit, CW).transpose(1, 2, 0, 3).reshape(nq, din, CW)
    aw = jnp.stack([a_s, a_d], axis=-1).reshape(H, nsplit, CW, 2).reshape(nq, CW, 2)
    return pl.pallas_call(
        functools.partial(_proj_body, nsplit),
        grid=(NBLK, nq),
        in_specs=[
            pl.BlockSpec((BLK, din), lambda i, q: (i, 0)),
            pl.BlockSpec((1, din, CW), lambda i, q: (q, 0, 0)),
            pl.BlockSpec((1, CW, 2), lambda i, q: (q, 0, 0)),
        ],
        out_specs=[
            pl.BlockSpec((1, BLK, CW), lambda i, q: (q, i, 0)),
            pl.BlockSpec((1, BLK, 2), lambda i, q: (q // nsplit, i, 0)),
        ],
        out_shape=[
            jax.ShapeDtypeStruct((nq, N, CW), jnp.float32),
            jax.ShapeDtypeStruct((H, N, 2), jnp.float32),
        ],
    )(x, wf, aw)


def _epilogue(agg_ref, b_ref, nqp):
    parts = []
    for qq in range(nqp):
        a = agg_ref[qq]
        den = jnp.broadcast_to(a[:, CW:CW + 1], (BLK, CW))
        parts.append(a[:, 0:CW] / (den + 1e-9))
    out = jnp.concatenate(parts, axis=-1) + b_ref[...]
    return jnp.maximum(out, 0.0)


def _fproj_body(nsplit, nqp, agg_ref, b_ref, w_ref, aw_ref, h_ref, al_ref):
    q = pl.program_id(1)
    x = _epilogue(agg_ref, b_ref, nqp)
    h = jnp.dot(x, w_ref[0], preferred_element_type=jnp.float32)
    h_ref[0] = h
    al = jnp.dot(h, aw_ref[0], preferred_element_type=jnp.float32)
    if nsplit == 1:
        al_ref[0] = al
    else:
        @pl.when(q % nsplit == 0)
        def _():
            al_ref[0] = al

        @pl.when(q % nsplit != 0)
        def _():
            al_ref[0] += al


def _fproj(agg, b, w, a_s, a_d):
    din, _, c = w.shape
    nqp = agg.shape[0]
    nsplit = c // CW
    nq = H * nsplit
    wf = w.reshape(din, H, nsplit, CW).transpose(1, 2, 0, 3).reshape(nq, din, CW)
    aw = jnp.stack([a_s, a_d], axis=-1).reshape(H, nsplit, CW, 2).reshape(nq, CW, 2)
    return pl.pallas_call(
        functools.partial(_fproj_body, nsplit, nqp),
        grid=(NBLK, nq),
        in_specs=[
            pl.BlockSpec((nqp, BLK, MW), lambda i, q: (0, i, 0)),
            pl.BlockSpec((1, din), lambda i, q: (0, 0)),
            pl.BlockSpec((1, din, CW), lambda i, q: (q, 0, 0)),
            pl.BlockSpec((1, CW, 2), lambda i, q: (q, 0, 0)),
        ],
        out_specs=[
            pl.BlockSpec((1, BLK, CW), lambda i, q: (q, i, 0)),
            pl.BlockSpec((1, BLK, 2), lambda i, q: (q // nsplit, i, 0)),
        ],
        out_shape=[
            jax.ShapeDtypeStruct((nq, N, CW), jnp.float32),
            jax.ShapeDtypeStruct((H, N, 2), jnp.float32),
        ],
    )(agg, b.reshape(1, din), wf, aw)


def _prep_body(al_ref, a_ref):
    al = al_ref[0]
    amax = jnp.max(al[:, 0:1], axis=0, keepdims=True)
    a_ref[...] = jnp.broadcast_to(amax, (1, 1, 16))


def _prep(al):
    return pl.pallas_call(
        _prep_body,
        grid=(H,),
        in_specs=[pl.BlockSpec((1, N, 2), lambda h: (h, 0, 0))],
        out_specs=pl.BlockSpec((1, 1, 16), lambda h: (h, 0, 0)),
        out_shape=jax.ShapeDtypeStruct((H, 1, 16), jnp.float32),
    )(al)


def _finish_body(agg_ref, b_ref, o_ref):
    o_ref[...] = _epilogue(agg_ref, b_ref, agg_ref.shape[0])


def _finish(agg, b):
    nq = agg.shape[0]
    hc = nq * CW
    return pl.pallas_call(
        _finish_body,
        grid=(NBLK,),
        in_specs=[
            pl.BlockSpec((nq, BLK, MW), lambda i: (0, i, 0)),
            pl.BlockSpec((1, hc), lambda i: (0, 0)),
        ],
        out_specs=pl.BlockSpec((BLK, hc), lambda i: (i, 0)),
        out_shape=jax.ShapeDtypeStruct((N, hc), jnp.float32),
    )(agg, b.reshape(1, hc))


def _fpool_body(agg1_ref, b1_ref, agg2_ref, b2_ref, o_ref):
    i = pl.program_id(0)
    x1 = _epilogue(agg1_ref, b1_ref, agg1_ref.shape[0])
    x2 = _epilogue(agg2_ref, b2_ref, agg2_ref.shape[0])
    s = jnp.concatenate([jnp.sum(x1, axis=0), jnp.sum(x2, axis=0)], axis=-1)

    @pl.when(i == 0)
    def _():
        o_ref[...] = jnp.zeros_like(o_ref)

    o_ref[0, :] += s


def _fpool(agg1, b1, agg2, b2):
    nq1 = agg1.shape[0]
    nq2 = agg2.shape[0]
    d = (nq1 + nq2) * CW
    out = pl.pallas_call(
        _fpool_body,
        grid=(NBLK,),
        in_specs=[
            pl.BlockSpec((nq1, BLK, MW), lambda i: (0, i, 0)),
            pl.BlockSpec((1, nq1 * CW), lambda i: (0, 0)),
            pl.BlockSpec((nq2, BLK, MW), lambda i: (0, i, 0)),
            pl.BlockSpec((1, nq2 * CW), lambda i: (0, 0)),
        ],
        out_specs=pl.BlockSpec((1, d), lambda i: (0, 0)),
        out_shape=jax.ShapeDtypeStruct((1, d), jnp.float32),
    )(agg1, b1.reshape(1, nq1 * CW), agg2, b2.reshape(1, nq2 * CW))
    return out[0]


NB = EPT // EB
PRO = 4


def _chunk_section(ch, hh, src_hbm, dst_hbm, t_hbm, a_hbm, h_hbm, out_hbm,
                   tloc, abuf, srcl, dstl, rows, msg, pb, zbuf, agg_sh, sid,
                   semg):
    pltpu.sync_copy(t_hbm.at[hh], tloc)
    pltpu.sync_copy(a_hbm.at[hh], abuf)
    for k in range(-(-NZBLK // NSUB)):
        zb = sid + k * NSUB

        @pl.when(zb < NZBLK)
        def _():
            pltpu.sync_copy(zbuf, agg_sh.at[pl.ds(zb * ZB, ZB)])
    plsc.subcore_barrier()

    @pl.loop(0, NB // QNB)
    def _(half):
        pltpu.sync_copy(src_hbm.at[sid].at[pl.ds(half * QNB, QNB)], srcl)
        pltpu.sync_copy(dst_hbm.at[sid].at[pl.ds(half * QNB, QNB)], dstl)

        def gsrc(lb):
            return h_hbm.at[ch].at[srcl.at[lb]]

        def sdst(lb):
            return agg_sh.at[dstl.at[lb]]

        pltpu.async_copy(gsrc(0), rows.at[0], semg.at[0])

        @pl.loop(0, QNB)
        def _(lb):
            r = lax.rem(lb, 2)
            nxt = 1 - r
            pltpu.make_async_copy(gsrc(lb), rows.at[r], semg.at[r]).wait()

            @pl.when(lb + 1 < QNB)
            def _():
                pltpu.async_copy(gsrc(lb + 1), rows.at[nxt], semg.at[nxt])

            av = abuf[0, pl.ds(0, 16)]
            for g in range(EB // 16):
                sv = srcl[lb, pl.ds(g * 16, 16)]
                dv = dstl[lb, pl.ds(g * 16, 16)]
                als = plsc.load_gather(
                    tloc, [sv, jnp.zeros((16,), jnp.int32)])
                ald = plsc.load_gather(
                    tloc, [dv, jnp.full((16,), 1, jnp.int32)])
                q = als + ald
                e = jnp.maximum(q, 0.2 * q)
                t = av + ald
                mm = jnp.maximum(t, 0.2 * t)
                p = jnp.exp(e - mm)
                pb[pl.ds(g * 16, 16)] = p
                plsc.store_scatter(
                    msg,
                    [lax.iota(jnp.int32, 16) + g * 16,
                     jnp.full((16,), CW, jnp.int32)],
                    p)

            @pl.loop(0, EB)
            def _(j):
                pj = plsc.load_gather(pb, [jnp.full((16,), j, jnp.int32)])
                for v in range(CW // 16):
                    msg[j, pl.ds(v * 16, 16)] = (
                        rows[r, j, pl.ds(v * 16, 16)] * pj)

            pltpu.sync_copy(msg, sdst(lb), add=True)

    plsc.subcore_barrier()
    for k in range(-(-NZBLK // NSUB)):
        zb = sid + k * NSUB

        @pl.when(zb < NZBLK)
        def _():
            pltpu.sync_copy(agg_sh.at[pl.ds(zb * ZB, ZB)],
                            out_hbm.at[ch].at[pl.ds(zb * ZB, ZB)])
    plsc.subcore_barrier()


def _edge_pair_body(nqa, nqb, src_hbm, dst_hbm, ta_hbm, aa_hbm, tb_hbm,
                    ab_hbm, ha_hbm, hb_hbm, outa_hbm, outb_hbm, tloc, abuf,
                    srcl, dstl, rows, msg, pb, zbuf, agg_sh, semg):
    cid = lax.axis_index("c")
    sid = lax.axis_index("s")
    zero16 = jnp.zeros((16,), jnp.float32)
    nsa = nqa // H

    @pl.loop(0, ZB)
    def _(r):
        for v in range(MW // 16):
            zbuf[r, pl.ds(v * 16, 16)] = zero16

    @pl.loop(0, EB)
    def _(r):
        msg[r, pl.ds(CW, 16)] = zero16

    for k in range(nqa // NCORE):
        ch = cid * (nqa // NCORE) + k
        hh = cid * (nqa // NCORE // nsa) + k // nsa
        _chunk_section(ch, hh, src_hbm, dst_hbm, ta_hbm, aa_hbm, ha_hbm,
                       outa_hbm, tloc, abuf, srcl, dstl, rows, msg, pb,
                       zbuf, agg_sh, sid, semg)
    for k in range(nqb // NCORE):
        ch = cid * (nqb // NCORE) + k
        _chunk_section(ch, ch, src_hbm, dst_hbm, tb_hbm, ab_hbm, hb_hbm,
                       outb_hbm, tloc, abuf, srcl, dstl, rows, msg, pb,
                       zbuf, agg_sh, sid, semg)


def _edge_pair(src3d, dst3d, ta, aa, tb, ab, ha, hb):
    nqa = ha.shape[0]
    nqb = hb.shape[0]
    mesh = plsc.VectorSubcoreMesh(core_axis_name="c", subcore_axis_name="s")
    fn = pl.kernel(
        functools.partial(_edge_pair_body, nqa, nqb),
        out_type=[
            jax.ShapeDtypeStruct((nqa, N, MW), jnp.float32),
            jax.ShapeDtypeStruct((nqb, N, MW), jnp.float32),
        ],
        mesh=mesh,
        compiler_params=pltpu.CompilerParams(
            needs_layout_passes=False, use_tc_tiling_on_sc=False),
        scratch_types=[
            pltpu.VMEM((N, 2), jnp.float32),
            pltpu.VMEM((1, 16), jnp.float32),
            pltpu.VMEM((QNB, EB), jnp.int32),
            pltpu.VMEM((QNB, EB), jnp.int32),
            pltpu.VMEM((2, EB, CW), jnp.float32),
            pltpu.VMEM((EB, MW), jnp.float32),
            pltpu.VMEM((EB,), jnp.float32),
            pltpu.VMEM((ZB, MW), jnp.float32),
            pltpu.VMEM_SHARED((N, MW), jnp.float32),
            pltpu.SemaphoreType.DMA((2,)),
        ],
    )
    return fn(src3d, dst3d, ta, aa, tb, ab, ha, hb)


def _edges(al_a, al_b, ha, hb, src, dst):
    return _edge_pair(src, dst, al_a, _prep(al_a), al_b, _prep(al_b), ha, hb)


def kernel(x, W1, a_src1, a_dst1, b1, W2, a_src2, a_dst2, b2, W3, a_src3, a_dst3, b3,
           W4, a_src4, a_dst4, b4, W5, a_src5, a_dst5, b5, W6, a_src6, a_dst6, b6,
           edge_index):
    src = edge_index[0].reshape(NSUB, NB, EB)
    dst = edge_index[1].reshape(NSUB, NB, EB)
    h1, al1 = _proj(x, W1, a_src1, a_dst1)
    h4, al4 = _proj(x, W4, a_src4, a_dst4)
    agg1, agg4 = _edges(al1, al4, h1, h4, src, dst)
    h2, al2 = _proj(_finish(agg1, b1), W2, a_src2, a_dst2)
    h5, al5 = _proj(_finish(agg4, b4), W5, a_src5, a_dst5)
    agg2, agg5 = _edges(al2, al5, h2, h5, src, dst)
    h3, al3 = _proj(_finish(agg2, b2), W3, a_src3, a_dst3)
    h6, al6 = _proj(_finish(agg5, b5), W6, a_src6, a_dst6)
    agg3, agg6 = _edges(al3, al6, h3, h6, src, dst)
    return _fpool(agg3, b3, agg6, b6)


# speedup vs baseline: 1.1375x; 1.0538x over previous
import functools

import jax
import jax.numpy as jnp
from jax import lax
from jax.experimental import pallas as pl
from jax.experimental.pallas import tpu as pltpu
from jax.experimental.pallas import tpu_sc as plsc

N = 10000
E = 320000
H = 8
CW = 32
MW = CW + 16
NSUB = 16
NCORE = 2
EB = 80
EPT = E // NSUB
ZB = 80
NZBLK = N // ZB
QNB = 50
NBLK = 10
BLK = N // NBLK


def _proj_body(nsplit, x_ref, w_ref, aw_ref, h_ref, al_ref):
    q = pl.program_id(1)
    h = jnp.dot(x_ref[...], w_ref[0], preferred_element_type=jnp.float32)
    h_ref[0] = h
    al = jnp.dot(h, aw_ref[0], preferred_element_type=jnp.float32)
    if nsplit == 1:
        al_ref[0] = al
    else:
        @pl.when(q % nsplit == 0)
        def _():
            al_ref[0] = al

        @pl.when(q % nsplit != 0)
        def _():
            al_ref[0] += al


def _proj(x, w, a_s, a_d):
    din, _, c = w.shape
    nsplit = c // CW
    nq = H * nsplit
    wf = w.reshape(din, H, nsplit, CW).transpose(1, 2, 0, 3).reshape(nq, din, CW)
    aw = jnp.stack([a_s, a_d], axis=-1).reshape(H, nsplit, CW, 2).reshape(nq, CW, 2)
    return pl.pallas_call(
        functools.partial(_proj_body, nsplit),
        grid=(NBLK, nq),
        in_specs=[
            pl.BlockSpec((BLK, din), lambda i, q: (i, 0)),
            pl.BlockSpec((1, din, CW), lambda i, q: (q, 0, 0)),
            pl.BlockSpec((1, CW, 2), lambda i, q: (q, 0, 0)),
        ],
        out_specs=[
            pl.BlockSpec((1, BLK, CW), lambda i, q: (q, i, 0)),
            pl.BlockSpec((1, BLK, 2), lambda i, q: (q // nsplit, i, 0)),
        ],
        out_shape=[
            jax.ShapeDtypeStruct((nq, N, CW), jnp.float32),
            jax.ShapeDtypeStruct((H, N, 2), jnp.float32),
        ],
    )(x, wf, aw)


def _epilogue(agg_ref, b_ref, nqp):
    parts = []
    for qq in range(nqp):
        a = agg_ref[qq]
        den = jnp.broadcast_to(a[:, CW:CW + 1], (BLK, CW))
        parts.append(a[:, 0:CW] / (den + 1e-9))
    out = jnp.concatenate(parts, axis=-1) + b_ref[...]
    return jnp.maximum(out, 0.0)


def _fproj_body(nsplit, nqp, agg_ref, b_ref, w_ref, aw_ref, h_ref, al_ref):
    q = pl.program_id(1)
    x = _epilogue(agg_ref, b_ref, nqp)
    h = jnp.dot(x, w_ref[0], preferred_element_type=jnp.float32)
    h_ref[0] = h
    al = jnp.dot(h, aw_ref[0], preferred_element_type=jnp.float32)
    if nsplit == 1:
        al_ref[0] = al
    else:
        @pl.when(q % nsplit == 0)
        def _():
            al_ref[0] = al

        @pl.when(q % nsplit != 0)
        def _():
            al_ref[0] += al


def _fproj(agg, b, w, a_s, a_d):
    din, _, c = w.shape
    nqp = agg.shape[0]
    nsplit = c // CW
    nq = H * nsplit
    wf = w.reshape(din, H, nsplit, CW).transpose(1, 2, 0, 3).reshape(nq, din, CW)
    aw = jnp.stack([a_s, a_d], axis=-1).reshape(H, nsplit, CW, 2).reshape(nq, CW, 2)
    return pl.pallas_call(
        functools.partial(_fproj_body, nsplit, nqp),
        grid=(NBLK, nq),
        in_specs=[
            pl.BlockSpec((nqp, BLK, MW), lambda i, q: (0, i, 0)),
            pl.BlockSpec((1, din), lambda i, q: (0, 0)),
            pl.BlockSpec((1, din, CW), lambda i, q: (q, 0, 0)),
            pl.BlockSpec((1, CW, 2), lambda i, q: (q, 0, 0)),
        ],
        out_specs=[
            pl.BlockSpec((1, BLK, CW), lambda i, q: (q, i, 0)),
            pl.BlockSpec((1, BLK, 2), lambda i, q: (q // nsplit, i, 0)),
        ],
        out_shape=[
            jax.ShapeDtypeStruct((nq, N, CW), jnp.float32),
            jax.ShapeDtypeStruct((H, N, 2), jnp.float32),
        ],
    )(agg, b.reshape(1, din), wf, aw)


def _prep_body(al_ref, a_ref):
    al = al_ref[0]
    amax = jnp.max(al[:, 0:1], axis=0, keepdims=True)
    a_ref[...] = jnp.broadcast_to(amax, (1, 1, 16))


def _prep(al):
    return pl.pallas_call(
        _prep_body,
        grid=(H,),
        in_specs=[pl.BlockSpec((1, N, 2), lambda h: (h, 0, 0))],
        out_specs=pl.BlockSpec((1, 1, 16), lambda h: (h, 0, 0)),
        out_shape=jax.ShapeDtypeStruct((H, 1, 16), jnp.float32),
    )(al)


def _finish_body(agg_ref, b_ref, o_ref):
    o_ref[...] = _epilogue(agg_ref, b_ref, agg_ref.shape[0])


def _finish(agg, b):
    nq = agg.shape[0]
    hc = nq * CW
    return pl.pallas_call(
        _finish_body,
        grid=(NBLK,),
        in_specs=[
            pl.BlockSpec((nq, BLK, MW), lambda i: (0, i, 0)),
            pl.BlockSpec((1, hc), lambda i: (0, 0)),
        ],
        out_specs=pl.BlockSpec((BLK, hc), lambda i: (i, 0)),
        out_shape=jax.ShapeDtypeStruct((N, hc), jnp.float32),
    )(agg, b.reshape(1, hc))


def _fpool_body(agg1_ref, b1_ref, agg2_ref, b2_ref, o_ref):
    i = pl.program_id(0)
    x1 = _epilogue(agg1_ref, b1_ref, agg1_ref.shape[0])
    x2 = _epilogue(agg2_ref, b2_ref, agg2_ref.shape[0])
    s = jnp.concatenate([jnp.sum(x1, axis=0), jnp.sum(x2, axis=0)], axis=-1)

    @pl.when(i == 0)
    def _():
        o_ref[...] = jnp.zeros_like(o_ref)

    o_ref[0, :] += s


def _fpool(agg1, b1, agg2, b2):
    nq1 = agg1.shape[0]
    nq2 = agg2.shape[0]
    d = (nq1 + nq2) * CW
    out = pl.pallas_call(
        _fpool_body,
        grid=(NBLK,),
        in_specs=[
            pl.BlockSpec((nq1, BLK, MW), lambda i: (0, i, 0)),
            pl.BlockSpec((1, nq1 * CW), lambda i: (0, 0)),
            pl.BlockSpec((nq2, BLK, MW), lambda i: (0, i, 0)),
            pl.BlockSpec((1, nq2 * CW), lambda i: (0, 0)),
        ],
        out_specs=pl.BlockSpec((1, d), lambda i: (0, 0)),
        out_shape=jax.ShapeDtypeStruct((1, d), jnp.float32),
    )(agg1, b1.reshape(1, nq1 * CW), agg2, b2.reshape(1, nq2 * CW))
    return out[0]


NB = EPT // EB
PRO = 4


def _chunk_section(ch, hh, src_hbm, dst_hbm, t_hbm, a_hbm, h_hbm, out_hbm,
                   tloc, abuf, srcl, dstl, rows, msg, pb, zbuf, agg_sh, sid,
                   semg):
    pltpu.sync_copy(t_hbm.at[hh], tloc)
    pltpu.sync_copy(a_hbm.at[hh], abuf)
    for k in range(-(-NZBLK // NSUB)):
        zb = sid + k * NSUB

        @pl.when(zb < NZBLK)
        def _():
            pltpu.sync_copy(zbuf, agg_sh.at[pl.ds(zb * ZB, ZB)])
    plsc.subcore_barrier()

    @pl.loop(0, NB // QNB)
    def _(half):
        pltpu.sync_copy(src_hbm.at[sid].at[pl.ds(half * QNB, QNB)], srcl)
        pltpu.sync_copy(dst_hbm.at[sid].at[pl.ds(half * QNB, QNB)], dstl)

        def gsrc(lb):
            return h_hbm.at[ch].at[srcl.at[lb]]

        def sdst(lb):
            return agg_sh.at[dstl.at[lb]]

        pltpu.async_copy(gsrc(0), rows.at[0], semg.at[0])

        @pl.loop(0, QNB)
        def _(lb):
            r = lax.rem(lb, 2)
            nxt = 1 - r
            pltpu.make_async_copy(gsrc(lb), rows.at[r], semg.at[r]).wait()

            @pl.when(lb + 1 < QNB)
            def _():
                pltpu.async_copy(gsrc(lb + 1), rows.at[nxt], semg.at[nxt])

            av = abuf[0, pl.ds(0, 16)]
            for g in range(EB // 16):
                sv = srcl[lb, pl.ds(g * 16, 16)]
                dv = dstl[lb, pl.ds(g * 16, 16)]
                als = plsc.load_gather(
                    tloc, [sv, jnp.zeros((16,), jnp.int32)])
                ald = plsc.load_gather(
                    tloc, [dv, jnp.full((16,), 1, jnp.int32)])
                q = als + ald
                e = jnp.maximum(q, 0.2 * q)
                t = av + ald
                mm = jnp.maximum(t, 0.2 * t)
                p = jnp.exp(e - mm)
                pb[pl.ds(g * 16, 16)] = p
                plsc.store_scatter(
                    msg,
                    [lax.iota(jnp.int32, 16) + g * 16,
                     jnp.full((16,), CW, jnp.int32)],
                    p)

            @pl.loop(0, EB, step=4)
            def _(j0):
                for dj in range(4):
                    j = j0 + dj
                    pj = plsc.load_gather(
                        pb, [jnp.full((16,), j, jnp.int32)])
                    for v in range(CW // 16):
                        msg[j, pl.ds(v * 16, 16)] = (
                            rows[r, j, pl.ds(v * 16, 16)] * pj)

            pltpu.sync_copy(msg, sdst(lb), add=True)

    plsc.subcore_barrier()
    for k in range(-(-NZBLK // NSUB)):
        zb = sid + k * NSUB

        @pl.when(zb < NZBLK)
        def _():
            pltpu.sync_copy(agg_sh.at[pl.ds(zb * ZB, ZB)],
                            out_hbm.at[ch].at[pl.ds(zb * ZB, ZB)])
    plsc.subcore_barrier()


def _edge_pair_body(nqa, nqb, src_hbm, dst_hbm, ta_hbm, aa_hbm, tb_hbm,
                    ab_hbm, ha_hbm, hb_hbm, outa_hbm, outb_hbm, tloc, abuf,
                    srcl, dstl, rows, msg, pb, zbuf, agg_sh, semg):
    cid = lax.axis_index("c")
    sid = lax.axis_index("s")
    zero16 = jnp.zeros((16,), jnp.float32)
    nsa = nqa // H

    @pl.loop(0, ZB)
    def _(r):
        for v in range(MW // 16):
            zbuf[r, pl.ds(v * 16, 16)] = zero16

    @pl.loop(0, EB)
    def _(r):
        msg[r, pl.ds(CW, 16)] = zero16

    for k in range(nqa // NCORE):
        ch = cid * (nqa // NCORE) + k
        hh = cid * (nqa // NCORE // nsa) + k // nsa
        _chunk_section(ch, hh, src_hbm, dst_hbm, ta_hbm, aa_hbm, ha_hbm,
                       outa_hbm, tloc, abuf, srcl, dstl, rows, msg, pb,
                       zbuf, agg_sh, sid, semg)
    for k in range(nqb // NCORE):
        ch = cid * (nqb // NCORE) + k
        _chunk_section(ch, ch, src_hbm, dst_hbm, tb_hbm, ab_hbm, hb_hbm,
                       outb_hbm, tloc, abuf, srcl, dstl, rows, msg, pb,
                       zbuf, agg_sh, sid, semg)


def _edge_pair(src3d, dst3d, ta, aa, tb, ab, ha, hb):
    nqa = ha.shape[0]
    nqb = hb.shape[0]
    mesh = plsc.VectorSubcoreMesh(core_axis_name="c", subcore_axis_name="s")
    fn = pl.kernel(
        functools.partial(_edge_pair_body, nqa, nqb),
        out_type=[
            jax.ShapeDtypeStruct((nqa, N, MW), jnp.float32),
            jax.ShapeDtypeStruct((nqb, N, MW), jnp.float32),
        ],
        mesh=mesh,
        compiler_params=pltpu.CompilerParams(
            needs_layout_passes=False, use_tc_tiling_on_sc=False),
        scratch_types=[
            pltpu.VMEM((N, 2), jnp.float32),
            pltpu.VMEM((1, 16), jnp.float32),
            pltpu.VMEM((QNB, EB), jnp.int32),
            pltpu.VMEM((QNB, EB), jnp.int32),
            pltpu.VMEM((2, EB, CW), jnp.float32),
            pltpu.VMEM((EB, MW), jnp.float32),
            pltpu.VMEM((EB,), jnp.float32),
            pltpu.VMEM((ZB, MW), jnp.float32),
            pltpu.VMEM_SHARED((N, MW), jnp.float32),
            pltpu.SemaphoreType.DMA((2,)),
        ],
    )
    return fn(src3d, dst3d, ta, aa, tb, ab, ha, hb)


def _edges(al_a, al_b, ha, hb, src, dst):
    return _edge_pair(src, dst, al_a, _prep(al_a), al_b, _prep(al_b), ha, hb)


def kernel(x, W1, a_src1, a_dst1, b1, W2, a_src2, a_dst2, b2, W3, a_src3, a_dst3, b3,
           W4, a_src4, a_dst4, b4, W5, a_src5, a_dst5, b5, W6, a_src6, a_dst6, b6,
           edge_index):
    src = edge_index[0].reshape(NSUB, NB, EB)
    dst = edge_index[1].reshape(NSUB, NB, EB)
    h1, al1 = _proj(x, W1, a_src1, a_dst1)
    h4, al4 = _proj(x, W4, a_src4, a_dst4)
    agg1, agg4 = _edges(al1, al4, h1, h4, src, dst)
    h2, al2 = _proj(_finish(agg1, b1), W2, a_src2, a_dst2)
    h5, al5 = _proj(_finish(agg4, b4), W5, a_src5, a_dst5)
    agg2, agg5 = _edges(al2, al5, h2, h5, src, dst)
    h3, al3 = _proj(_finish(agg2, b2), W3, a_src3, a_dst3)
    h6, al6 = _proj(_finish(agg5, b5), W6, a_src6, a_dst6)
    agg3, agg6 = _edges(al3, al6, h3, h6, src, dst)
    return _fpool(agg3, b3, agg6, b6)


# speedup vs baseline: 1.1378x; 1.0002x over previous
import functools

import jax
import jax.numpy as jnp
from jax import lax
from jax.experimental import pallas as pl
from jax.experimental.pallas import tpu as pltpu
from jax.experimental.pallas import tpu_sc as plsc

N = 10000
E = 320000
H = 8
CW = 32
MW = CW + 16
NSUB = 16
NCORE = 2
EB = 80
EPT = E // NSUB
ZB = 80
NZBLK = N // ZB
QNB = 50
NBLK = 10
BLK = N // NBLK


def _proj_body(nsplit, x_ref, w_ref, aw_ref, h_ref, al_ref):
    q = pl.program_id(1)
    h = jnp.dot(x_ref[...], w_ref[0], preferred_element_type=jnp.float32)
    h_ref[0] = h
    al = jnp.dot(h, aw_ref[0], preferred_element_type=jnp.float32)
    if nsplit == 1:
        al_ref[0] = al
    else:
        @pl.when(q % nsplit == 0)
        def _():
            al_ref[0] = al

        @pl.when(q % nsplit != 0)
        def _():
            al_ref[0] += al


def _proj(x, w, a_s, a_d):
    din, _, c = w.shape
    nsplit = c // CW
    nq = H * nsplit
    wf = w.reshape(din, H, nsplit, CW).transpose(1, 2, 0, 3).reshape(nq, din, CW)
    aw = jnp.stack([a_s, a_d], axis=-1).reshape(H, nsplit, CW, 2).reshape(nq, CW, 2)
    return pl.pallas_call(
        functools.partial(_proj_body, nsplit),
        grid=(NBLK, nq),
        in_specs=[
            pl.BlockSpec((BLK, din), lambda i, q: (i, 0)),
            pl.BlockSpec((1, din, CW), lambda i, q: (q, 0, 0)),
            pl.BlockSpec((1, CW, 2), lambda i, q: (q, 0, 0)),
        ],
        out_specs=[
            pl.BlockSpec((1, BLK, CW), lambda i, q: (q, i, 0)),
            pl.BlockSpec((1, BLK, 2), lambda i, q: (q // nsplit, i, 0)),
        ],
        out_shape=[
            jax.ShapeDtypeStruct((nq, N, CW), jnp.float32),
            jax.ShapeDtypeStruct((H, N, 2), jnp.float32),
        ],
    )(x, wf, aw)


def _epilogue(agg_ref, b_ref, nqp):
    parts = []
    for qq in range(nqp):
        a = agg_ref[qq]
        den = jnp.broadcast_to(a[:, CW:CW + 1], (BLK, CW))
        parts.append(a[:, 0:CW] / (den + 1e-9))
    out = jnp.concatenate(parts, axis=-1) + b_ref[...]
    return jnp.maximum(out, 0.0)


def _prep_body(al_ref, a_ref):
    al = al_ref[0]
    amax = jnp.max(al[:, 0:1], axis=0, keepdims=True)
    a_ref[...] = jnp.broadcast_to(amax, (1, 1, 16))


def _prep(al):
    return pl.pallas_call(
        _prep_body,
        grid=(H,),
        in_specs=[pl.BlockSpec((1, N, 2), lambda h: (h, 0, 0))],
        out_specs=pl.BlockSpec((1, 1, 16), lambda h: (h, 0, 0)),
        out_shape=jax.ShapeDtypeStruct((H, 1, 16), jnp.float32),
    )(al)


def _finish_body(agg_ref, b_ref, o_ref):
    o_ref[...] = _epilogue(agg_ref, b_ref, agg_ref.shape[0])


def _finish(agg, b):
    nq = agg.shape[0]
    hc = nq * CW
    return pl.pallas_call(
        _finish_body,
        grid=(NBLK,),
        in_specs=[
            pl.BlockSpec((nq, BLK, MW), lambda i: (0, i, 0)),
            pl.BlockSpec((1, hc), lambda i: (0, 0)),
        ],
        out_specs=pl.BlockSpec((BLK, hc), lambda i: (i, 0)),
        out_shape=jax.ShapeDtypeStruct((N, hc), jnp.float32),
    )(agg, b.reshape(1, hc))


def _fpool_body(agg1_ref, b1_ref, agg2_ref, b2_ref, o_ref):
    i = pl.program_id(0)
    x1 = _epilogue(agg1_ref, b1_ref, agg1_ref.shape[0])
    x2 = _epilogue(agg2_ref, b2_ref, agg2_ref.shape[0])
    s = jnp.concatenate([jnp.sum(x1, axis=0), jnp.sum(x2, axis=0)], axis=-1)

    @pl.when(i == 0)
    def _():
        o_ref[...] = jnp.zeros_like(o_ref)

    o_ref[0, :] += s


def _fpool(agg1, b1, agg2, b2):
    nq1 = agg1.shape[0]
    nq2 = agg2.shape[0]
    d = (nq1 + nq2) * CW
    out = pl.pallas_call(
        _fpool_body,
        grid=(NBLK,),
        in_specs=[
            pl.BlockSpec((nq1, BLK, MW), lambda i: (0, i, 0)),
            pl.BlockSpec((1, nq1 * CW), lambda i: (0, 0)),
            pl.BlockSpec((nq2, BLK, MW), lambda i: (0, i, 0)),
            pl.BlockSpec((1, nq2 * CW), lambda i: (0, 0)),
        ],
        out_specs=pl.BlockSpec((1, d), lambda i: (0, 0)),
        out_shape=jax.ShapeDtypeStruct((1, d), jnp.float32),
    )(agg1, b1.reshape(1, nq1 * CW), agg2, b2.reshape(1, nq2 * CW))
    return out[0]


NB = EPT // EB


def _chunk_section(ch, hh, src_hbm, dst_hbm, t_hbm, a_hbm, h_hbm, out_hbm,
                   tloc, abuf, srcl, dstl, rows, msg, pb, zbuf, agg_sh, sid,
                   semg):
    pltpu.sync_copy(t_hbm.at[hh], tloc)
    pltpu.sync_copy(a_hbm.at[hh], abuf)
    for k in range(-(-NZBLK // NSUB)):
        zb = sid + k * NSUB

        @pl.when(zb < NZBLK)
        def _():
            pltpu.sync_copy(zbuf, agg_sh.at[pl.ds(zb * ZB, ZB)])
    plsc.subcore_barrier()

    @pl.loop(0, NB // QNB)
    def _(half):
        pltpu.sync_copy(src_hbm.at[sid].at[pl.ds(half * QNB, QNB)], srcl)
        pltpu.sync_copy(dst_hbm.at[sid].at[pl.ds(half * QNB, QNB)], dstl)

        def gsrc(lb):
            return h_hbm.at[ch].at[srcl.at[lb]]

        def sdst(lb):
            return agg_sh.at[dstl.at[lb]]

        pltpu.async_copy(gsrc(0), rows.at[0], semg.at[0])

        @pl.loop(0, QNB)
        def _(lb):
            r = lax.rem(lb, 2)
            nxt = 1 - r
            pltpu.make_async_copy(gsrc(lb), rows.at[r], semg.at[r]).wait()

            @pl.when(lb + 1 < QNB)
            def _():
                pltpu.async_copy(gsrc(lb + 1), rows.at[nxt], semg.at[nxt])

            av = abuf[0, pl.ds(0, 16)]
            for g in range(EB // 16):
                sv = srcl[lb, pl.ds(g * 16, 16)]
                dv = dstl[lb, pl.ds(g * 16, 16)]
                als = plsc.load_gather(
                    tloc, [sv, jnp.zeros((16,), jnp.int32)])
                ald = plsc.load_gather(
                    tloc, [dv, jnp.full((16,), 1, jnp.int32)])
                q = als + ald
                e = jnp.maximum(q, 0.2 * q)
                t = av + ald
                mm = jnp.maximum(t, 0.2 * t)
                p = jnp.exp(e - mm)
                pb[pl.ds(g * 16, 16)] = p
                plsc.store_scatter(
                    msg,
                    [lax.iota(jnp.int32, 16) + g * 16,
                     jnp.full((16,), CW, jnp.int32)],
                    p)

            @pl.loop(0, EB, step=4)
            def _(j0):
                for dj in range(4):
                    j = j0 + dj
                    pj = plsc.load_gather(
                        pb, [jnp.full((16,), j, jnp.int32)])
                    for v in range(CW // 16):
                        msg[j, pl.ds(v * 16, 16)] = (
                            rows[r, j, pl.ds(v * 16, 16)] * pj)

            pltpu.sync_copy(msg, sdst(lb), add=True)

    plsc.subcore_barrier()
    for k in range(-(-NZBLK // NSUB)):
        zb = sid + k * NSUB

        @pl.when(zb < NZBLK)
        def _():
            pltpu.sync_copy(agg_sh.at[pl.ds(zb * ZB, ZB)],
                            out_hbm.at[ch].at[pl.ds(zb * ZB, ZB)])
    plsc.subcore_barrier()


def _edge_pair_body(nqa, nqb, src_hbm, dst_hbm, ta_hbm, aa_hbm, tb_hbm,
                    ab_hbm, ha_hbm, hb_hbm, outa_hbm, outb_hbm, tloc, abuf,
                    srcl, dstl, rows, msg, pb, zbuf, agg_sh, semg):
    cid = lax.axis_index("c")
    sid = lax.axis_index("s")
    zero16 = jnp.zeros((16,), jnp.float32)
    nsa = nqa // H

    @pl.loop(0, ZB)
    def _(r):
        for v in range(MW // 16):
            zbuf[r, pl.ds(v * 16, 16)] = zero16

    @pl.loop(0, EB)
    def _(r):
        msg[r, pl.ds(CW, 16)] = zero16

    for k in range(nqa // NCORE):
        ch = cid * (nqa // NCORE) + k
        hh = cid * (nqa // NCORE // nsa) + k // nsa
        _chunk_section(ch, hh, src_hbm, dst_hbm, ta_hbm, aa_hbm, ha_hbm,
                       outa_hbm, tloc, abuf, srcl, dstl, rows, msg, pb,
                       zbuf, agg_sh, sid, semg)
    for k in range(nqb // NCORE):
        ch = cid * (nqb // NCORE) + k
        _chunk_section(ch, ch, src_hbm, dst_hbm, tb_hbm, ab_hbm, hb_hbm,
                       outb_hbm, tloc, abuf, srcl, dstl, rows, msg, pb,
                       zbuf, agg_sh, sid, semg)


def _edge_pair(src3d, dst3d, ta, aa, tb, ab, ha, hb):
    nqa = ha.shape[0]
    nqb = hb.shape[0]
    mesh = plsc.VectorSubcoreMesh(core_axis_name="c", subcore_axis_name="s")
    fn = pl.kernel(
        functools.partial(_edge_pair_body, nqa, nqb),
        out_type=[
            jax.ShapeDtypeStruct((nqa, N, MW), jnp.float32),
            jax.ShapeDtypeStruct((nqb, N, MW), jnp.float32),
        ],
        mesh=mesh,
        compiler_params=pltpu.CompilerParams(
            needs_layout_passes=False, use_tc_tiling_on_sc=False),
        scratch_types=[
            pltpu.VMEM((N, 2), jnp.float32),
            pltpu.VMEM((1, 16), jnp.float32),
            pltpu.VMEM((QNB, EB), jnp.int32),
            pltpu.VMEM((QNB, EB), jnp.int32),
            pltpu.VMEM((2, EB, CW), jnp.float32),
            pltpu.VMEM((EB, MW), jnp.float32),
            pltpu.VMEM((EB,), jnp.float32),
            pltpu.VMEM((ZB, MW), jnp.float32),
            pltpu.VMEM_SHARED((N, MW), jnp.float32),
            pltpu.SemaphoreType.DMA((2,)),
        ],
    )
    return fn(src3d, dst3d, ta, aa, tb, ab, ha, hb)


def _edges(al_a, al_b, ha, hb, src, dst):
    return _edge_pair(src, dst, al_a, _prep(al_a), al_b, _prep(al_b), ha, hb)


def kernel(x, W1, a_src1, a_dst1, b1, W2, a_src2, a_dst2, b2, W3, a_src3, a_dst3, b3,
           W4, a_src4, a_dst4, b4, W5, a_src5, a_dst5, b5, W6, a_src6, a_dst6, b6,
           edge_index):
    src = edge_index[0].reshape(NSUB, NB, EB)
    dst = edge_index[1].reshape(NSUB, NB, EB)
    h1, al1 = _proj(x, W1, a_src1, a_dst1)
    h4, al4 = _proj(x, W4, a_src4, a_dst4)
    agg1, agg4 = _edges(al1, al4, h1, h4, src, dst)
    h2, al2 = _proj(_finish(agg1, b1), W2, a_src2, a_dst2)
    h5, al5 = _proj(_finish(agg4, b4), W5, a_src5, a_dst5)
    agg2, agg5 = _edges(al2, al5, h2, h5, src, dst)
    h3, al3 = _proj(_finish(agg2, b2), W3, a_src3, a_dst3)
    h6, al6 = _proj(_finish(agg5, b5), W6, a_src6, a_dst6)
    agg3, agg6 = _edges(al3, al6, h3, h6, src, dst)
    return _fpool(agg3, b3, agg6, b6)
